# baseline (speedup 1.0000x reference)
import functools

import jax
import jax.numpy as jnp
from jax import lax
from jax.experimental import pallas as pl
from jax.experimental.pallas import tpu as pltpu
from jax.experimental.pallas import tpu_sc as plsc

VOCAB = 100000
EMB = 64
HID = 128
B = 1024
CTX = 20

NUM_CORES = 2
NUM_SUBCORES = 16
NW = NUM_CORES * NUM_SUBCORES
B_PER_W = B // NW
IDX_PER_W = B_PER_W * CTX
LANES = 16
EMB_CHUNKS = EMB // LANES

V_TILE = 2048
N_TILES = (VOCAB + V_TILE - 1) // V_TILE
LAST_OFF = VOCAB - V_TILE
NBUF = 4


def _gather_sum_body(idx_hbm, table_hbm, out_hbm, idx_v, rows_v, acc_v, sem):
    wid = lax.axis_index("s") * NUM_CORES + lax.axis_index("c")
    base = wid * IDX_PER_W
    pltpu.sync_copy(idx_hbm.at[pl.ds(base, IDX_PER_W)], idx_v)
    pltpu.async_copy(table_hbm.at[idx_v], rows_v, sem).wait()

    def row_body(r, _):
        rb = r * CTX
        for c in range(EMB_CHUNKS):
            s = rows_v[rb, pl.ds(c * LANES, LANES)]
            for t in range(1, CTX):
                s = s + rows_v[rb + t, pl.ds(c * LANES, LANES)]
            acc_v[r, pl.ds(c * LANES, LANES)] = s
        return 0

    lax.fori_loop(0, B_PER_W, row_body, 0)
    pltpu.sync_copy(acc_v, out_hbm.at[pl.ds(wid * B_PER_W, B_PER_W)])


@functools.cache
def _gather_sum_kernel():
    return pl.kernel(
        _gather_sum_body,
        mesh=plsc.VectorSubcoreMesh(core_axis_name="c", subcore_axis_name="s"),
        out_type=jax.ShapeDtypeStruct((B, EMB), jnp.float32),
        scratch_types=[
            pltpu.VMEM((IDX_PER_W,), jnp.int32),
            pltpu.VMEM((IDX_PER_W, EMB), jnp.float32),
            pltpu.VMEM((B_PER_W, EMB), jnp.float32),
            pltpu.SemaphoreType.DMA,
        ],
        compiler_params=pltpu.CompilerParams(use_tc_tiling_on_sc=False),
    )


def _out_off(i):
    return pl.multiple_of(jnp.minimum(i, N_TILES - 2) * V_TILE, V_TILE)


def _mlp_body(sum_ref, wh_ref, bh_ref, wo_ref, bo_ref, wo_last_ref,
              bo_last_ref, out_hbm, ob0, ob1, ob2, ob3, h_ref, sems):
    i = pl.program_id(0)
    obufs = (ob0, ob1, ob2, ob3)

    @pl.when(i == 0)
    def _():
        avg = sum_ref[...] * (1.0 / CTX)
        h = lax.dot_general(avg, wh_ref[...], (((1,), (1,)), ((), ())),
                            preferred_element_type=jnp.float32)
        h_ref[...] = jnp.maximum(h + bh_ref[...], 0.0)

    slot = lax.rem(i, NBUF)

    is_last = i == N_TILES - 1
    wo = jnp.where(is_last, wo_last_ref[...], wo_ref[...])
    bo = jnp.where(is_last, bo_last_ref[...], bo_ref[...])
    logits = lax.dot_general(h_ref[...], wo, (((1,), (1,)), ((), ())),
                             preferred_element_type=jnp.float32)
    val = logits + bo

    for k in range(NBUF):
        sel = slot == k

        @pl.when(jnp.logical_and(sel, i >= NBUF))
        def _(k=k):
            pltpu.make_async_copy(
                obufs[k],
                out_hbm.at[:, pl.ds(_out_off(i - NBUF), V_TILE)],
                sems.at[k],
            ).wait()

        @pl.when(sel)
        def _(k=k):
            obufs[k][...] = val
            pltpu.make_async_copy(
                obufs[k],
                out_hbm.at[:, pl.ds(_out_off(i), V_TILE)],
                sems.at[k],
            ).start()

    @pl.when(is_last)
    def _():
        for j in range(NBUF - 1, -1, -1):
            step = N_TILES - 1 - j
            k = step % NBUF
            pltpu.make_async_copy(
                obufs[k],
                out_hbm.at[:, pl.ds(_out_off(step), V_TILE)],
                sems.at[k],
            ).wait()


def _mlp(emb_sum, W_hidden, b_hidden, W_out, b_out):
    b_out2 = b_out.reshape(1, VOCAB)
    return pl.pallas_call(
        _mlp_body,
        grid=(N_TILES,),
        in_specs=[
            pl.BlockSpec((B, EMB), lambda i: (0, 0)),
            pl.BlockSpec((HID, EMB), lambda i: (0, 0)),
            pl.BlockSpec((1, HID), lambda i: (0, 0)),
            pl.BlockSpec((V_TILE, HID), lambda i: (jnp.minimum(i, N_TILES - 2), 0)),
            pl.BlockSpec((1, V_TILE), lambda i: (0, jnp.minimum(i, N_TILES - 2))),
            pl.BlockSpec((V_TILE, HID), lambda i: (0, 0)),
            pl.BlockSpec((1, V_TILE), lambda i: (0, 0)),
        ],
        out_specs=pl.BlockSpec(memory_space=pl.ANY),
        out_shape=jax.ShapeDtypeStruct((B, VOCAB), jnp.float32),
        scratch_shapes=[
            pltpu.VMEM((B, V_TILE), jnp.float32),
            pltpu.VMEM((B, V_TILE), jnp.float32),
            pltpu.VMEM((B, V_TILE), jnp.float32),
            pltpu.VMEM((B, V_TILE), jnp.float32),
            pltpu.VMEM((B, HID), jnp.float32),
            pltpu.SemaphoreType.DMA((NBUF,)),
        ],
    )(emb_sum, W_hidden, b_hidden.reshape(1, HID), W_out, b_out2,
      W_out[LAST_OFF:], b_out2[:, LAST_OFF:])


def kernel(context, emb_table, W_hidden, b_hidden, W_out, b_out):
    idx = context.reshape(-1).astype(jnp.int32)
    emb_sum = _gather_sum_kernel()(idx, emb_table)
    return _mlp(emb_sum, W_hidden, b_hidden, W_out, b_out)


# speedup vs baseline: 1.0163x; 1.0163x over previous
import functools

import jax
import jax.numpy as jnp
from jax import lax
from jax.experimental import pallas as pl
from jax.experimental.pallas import tpu as pltpu
from jax.experimental.pallas import tpu_sc as plsc

VOCAB = 100000
EMB = 64
HID = 128
B = 1024
CTX = 20

NUM_CORES = 2
NUM_SUBCORES = 16
NW = NUM_CORES * NUM_SUBCORES
B_PER_W = B // NW
IDX_PER_W = B_PER_W * CTX
LANES = 16
EMB_CHUNKS = EMB // LANES
EMB_PAD = 128

V_TILE = 2048
N_FULL = VOCAB // V_TILE
TAIL_OFF = N_FULL * V_TILE
TAIL = VOCAB - TAIL_OFF
N_TILES = N_FULL + 1
NBUF = 4


def _gather_sum_body(idx_hbm, table_hbm, out_hbm, idx_v, rows_v, acc_v, sem):
    wid = lax.axis_index("s") * NUM_CORES + lax.axis_index("c")
    base = wid * IDX_PER_W
    pltpu.sync_copy(idx_hbm.at[pl.ds(base, IDX_PER_W)], idx_v)
    pltpu.async_copy(table_hbm.at[idx_v], rows_v, sem).wait()

    def row_body(r, _):
        rb = r * CTX
        for c in range(EMB_CHUNKS):
            s = rows_v[rb, pl.ds(c * LANES, LANES)]
            for t in range(1, CTX):
                s = s + rows_v[rb + t, pl.ds(c * LANES, LANES)]
            acc_v[r, pl.ds(c * LANES, LANES)] = s
        return 0

    lax.fori_loop(0, B_PER_W, row_body, 0)
    pltpu.sync_copy(acc_v, out_hbm.at[pl.ds(wid * B_PER_W, B_PER_W)])


@functools.cache
def _gather_sum_kernel():
    return pl.kernel(
        _gather_sum_body,
        mesh=plsc.VectorSubcoreMesh(core_axis_name="c", subcore_axis_name="s"),
        out_type=jax.ShapeDtypeStruct((B, EMB_PAD), jnp.float32),
        scratch_types=[
            pltpu.VMEM((IDX_PER_W,), jnp.int32),
            pltpu.VMEM((IDX_PER_W, EMB_PAD), jnp.float32),
            pltpu.VMEM((B_PER_W, EMB_PAD), jnp.float32),
            pltpu.SemaphoreType.DMA,
        ],
    )


def _mlp_body(sum_ref, wh_ref, bh_ref, wo_ref, bo_ref, out_hbm,
              obuf, tbuf, h_ref, sems):
    i = pl.program_id(0)

    @pl.when(i == 0)
    def _():
        avg = sum_ref[...][:, :EMB] * (1.0 / CTX)
        h = lax.dot_general(avg, wh_ref[...], (((1,), (1,)), ((), ())),
                            preferred_element_type=jnp.float32)
        h_ref[...] = jnp.maximum(h + bh_ref[...], 0.0)

    slot = lax.rem(i, NBUF)

    @pl.when(i >= NBUF)
    def _():
        pltpu.make_async_copy(
            obuf.at[slot],
            out_hbm.at[:, pl.ds(pl.multiple_of((i - NBUF) * V_TILE, V_TILE),
                                V_TILE)],
            sems.at[slot],
        ).wait()

    logits = lax.dot_general(h_ref[...], wo_ref[...], (((1,), (1,)), ((), ())),
                             preferred_element_type=jnp.float32)
    val = logits + bo_ref[...]

    @pl.when(i < N_FULL)
    def _():
        obuf[slot] = val
        pltpu.make_async_copy(
            obuf.at[slot],
            out_hbm.at[:, pl.ds(pl.multiple_of(i * V_TILE, V_TILE), V_TILE)],
            sems.at[slot],
        ).start()

    @pl.when(i == N_FULL)
    def _():
        tbuf[...] = val[:, :TAIL]
        tail_copy = pltpu.make_async_copy(
            tbuf,
            out_hbm.at[:, pl.ds(TAIL_OFF, TAIL)],
            sems.at[N_FULL % NBUF],
        )
        tail_copy.start()
        for j in range(1, NBUF):
            step = N_FULL - j
            pltpu.make_async_copy(
                obuf.at[step % NBUF],
                out_hbm.at[:, pl.ds(pl.multiple_of(step * V_TILE, V_TILE),
                                    V_TILE)],
                sems.at[step % NBUF],
            ).wait()
        tail_copy.wait()


def _mlp(emb_sum, W_hidden, b_hidden, W_out, b_out):
    return pl.pallas_call(
        _mlp_body,
        grid=(N_TILES,),
        in_specs=[
            pl.BlockSpec((B, EMB_PAD), lambda i: (0, 0)),
            pl.BlockSpec((HID, EMB), lambda i: (0, 0)),
            pl.BlockSpec((1, HID), lambda i: (0, 0)),
            pl.BlockSpec((V_TILE, HID), lambda i: (i, 0)),
            pl.BlockSpec((1, V_TILE), lambda i: (0, i)),
        ],
        out_specs=pl.BlockSpec(memory_space=pl.ANY),
        out_shape=jax.ShapeDtypeStruct((B, VOCAB), jnp.float32),
        scratch_shapes=[
            pltpu.VMEM((NBUF, B, V_TILE), jnp.float32),
            pltpu.VMEM((B, TAIL), jnp.float32),
            pltpu.VMEM((B, HID), jnp.float32),
            pltpu.SemaphoreType.DMA((NBUF,)),
        ],
    )(emb_sum, W_hidden, b_hidden.reshape(1, HID), W_out,
      b_out.reshape(1, VOCAB))


def kernel(context, emb_table, W_hidden, b_hidden, W_out, b_out):
    idx = context.reshape(-1).astype(jnp.int32)
    table_pad = jnp.pad(emb_table, ((0, 0), (0, EMB_PAD - EMB)))
    emb_sum = _gather_sum_kernel()(idx, table_pad)
    return _mlp(emb_sum, W_hidden, b_hidden, W_out, b_out)


# speedup vs baseline: 1.0179x; 1.0015x over previous
import functools

import jax
import jax.numpy as jnp
from jax import lax
from jax.experimental import pallas as pl
from jax.experimental.pallas import tpu as pltpu
from jax.experimental.pallas import tpu_sc as plsc

VOCAB = 100000
EMB = 64
HID = 128
B = 1024
CTX = 20

NUM_CORES = 2
NUM_SUBCORES = 16
NW = NUM_CORES * NUM_SUBCORES
B_PER_W = B // NW
IDX_PER_W = B_PER_W * CTX
LANES = 16
EMB_CHUNKS = EMB // LANES
EMB_PAD = 128

V_TILE = 2048
N_FULL = VOCAB // V_TILE
TAIL_OFF = N_FULL * V_TILE
TAIL = VOCAB - TAIL_OFF
N_TILES = N_FULL + 1
NBUF = 4


def _gather_sum_body(idx_hbm, table_hbm, out_hbm, idx_v, rows_v, acc_v, sem):
    wid = lax.axis_index("s") * NUM_CORES + lax.axis_index("c")
    base = wid * IDX_PER_W
    pltpu.sync_copy(idx_hbm.at[pl.ds(base, IDX_PER_W)], idx_v)
    pltpu.async_copy(table_hbm.at[idx_v], rows_v, sem).wait()

    def row_body(r, _):
        rb = r * CTX
        for c in range(EMB_CHUNKS):
            s = rows_v[rb, pl.ds(c * LANES, LANES)]
            for t in range(1, CTX):
                s = s + rows_v[rb + t, pl.ds(c * LANES, LANES)]
            acc_v[r, pl.ds(c * LANES, LANES)] = s
        return 0

    lax.fori_loop(0, B_PER_W, row_body, 0)
    pltpu.sync_copy(acc_v, out_hbm.at[pl.ds(wid * B_PER_W, B_PER_W)])


@functools.cache
def _gather_sum_kernel():
    return pl.kernel(
        _gather_sum_body,
        mesh=plsc.VectorSubcoreMesh(core_axis_name="c", subcore_axis_name="s"),
        out_type=jax.ShapeDtypeStruct((B, EMB_PAD), jnp.float32),
        scratch_types=[
            pltpu.VMEM((IDX_PER_W,), jnp.int32),
            pltpu.VMEM((IDX_PER_W, EMB_PAD), jnp.float32),
            pltpu.VMEM((B_PER_W, EMB_PAD), jnp.float32),
            pltpu.SemaphoreType.DMA,
        ],
    )


def _mlp_body(sum_ref, wh_ref, bh_ref, wo_ref, bo_ref, out_hbm,
              obuf, tbuf, h_ref, sems):
    i = pl.program_id(0)

    @pl.when(i == 0)
    def _():
        avg = sum_ref[...][:, :EMB] * (1.0 / CTX)
        h = lax.dot_general(avg, wh_ref[...], (((1,), (1,)), ((), ())),
                            preferred_element_type=jnp.float32)
        h_ref[...] = jnp.maximum(h + bh_ref[...], 0.0)

    slot = lax.rem(i, NBUF)

    @pl.when(i >= NBUF)
    def _():
        pltpu.make_async_copy(
            obuf.at[slot],
            out_hbm.at[:, pl.ds(pl.multiple_of((i - NBUF) * V_TILE, V_TILE),
                                V_TILE)],
            sems.at[slot],
        ).wait()

    logits = lax.dot_general(h_ref[...], wo_ref[...], (((1,), (1,)), ((), ())),
                             preferred_element_type=jnp.float32)
    val = logits + bo_ref[...]

    @pl.when(i < N_FULL)
    def _():
        obuf[slot] = val
        pltpu.make_async_copy(
            obuf.at[slot],
            out_hbm.at[:, pl.ds(pl.multiple_of(i * V_TILE, V_TILE), V_TILE)],
            sems.at[slot],
        ).start()

    @pl.when(i == N_FULL)
    def _():
        tbuf[...] = val[:, :TAIL]
        tail_copy = pltpu.make_async_copy(
            tbuf,
            out_hbm.at[:, pl.ds(TAIL_OFF, TAIL)],
            sems.at[N_FULL % NBUF],
        )
        tail_copy.start()
        for j in range(1, NBUF):
            step = N_FULL - j
            pltpu.make_async_copy(
                obuf.at[step % NBUF],
                out_hbm.at[:, pl.ds(pl.multiple_of(step * V_TILE, V_TILE),
                                    V_TILE)],
                sems.at[step % NBUF],
            ).wait()
        tail_copy.wait()


def _mlp(emb_sum, W_hidden, b_hidden, W_out, b_out):
    return pl.pallas_call(
        _mlp_body,
        grid=(N_TILES,),
        in_specs=[
            pl.BlockSpec((B, EMB_PAD), lambda i: (0, 0)),
            pl.BlockSpec((HID, EMB), lambda i: (0, 0)),
            pl.BlockSpec((1, HID), lambda i: (0, 0)),
            pl.BlockSpec((V_TILE, HID), lambda i: (i, 0)),
            pl.BlockSpec((1, V_TILE), lambda i: (0, i)),
        ],
        out_specs=pl.BlockSpec(memory_space=pltpu.MemorySpace.HBM),
        out_shape=jax.ShapeDtypeStruct((B, VOCAB), jnp.float32),
        scratch_shapes=[
            pltpu.VMEM((NBUF, B, V_TILE), jnp.float32),
            pltpu.VMEM((B, TAIL), jnp.float32),
            pltpu.VMEM((B, HID), jnp.float32),
            pltpu.SemaphoreType.DMA((NBUF,)),
        ],
    )(emb_sum, W_hidden, b_hidden.reshape(1, HID), W_out,
      b_out.reshape(1, VOCAB))


def kernel(context, emb_table, W_hidden, b_hidden, W_out, b_out):
    idx = context.reshape(-1).astype(jnp.int32)
    table_pad = jnp.pad(emb_table, ((0, 0), (0, EMB_PAD - EMB)))
    emb_sum = _gather_sum_kernel()(idx, table_pad)
    return _mlp(emb_sum, W_hidden, b_hidden, W_out, b_out)


# speedup vs baseline: 2.0443x; 2.0083x over previous
import functools

import jax
import jax.numpy as jnp
from jax import lax
from jax.experimental import pallas as pl
from jax.experimental.pallas import tpu as pltpu
from jax.experimental.pallas import tpu_sc as plsc

VOCAB = 100000
EMB = 64
HID = 128
B = 1024
CTX = 20

NUM_CORES = 2
NUM_SUBCORES = 16
NW = NUM_CORES * NUM_SUBCORES
B_PER_W = B // NW
IDX_PER_W = B_PER_W * CTX
LANES = 16
EMB_CHUNKS = EMB // LANES

V_TILE = 2048
N_TILES = (VOCAB + V_TILE - 1) // V_TILE


def _gather_sum_body(idx_hbm, table_hbm, out_hbm, idx_v, rows_v, acc_v, sem):
    wid = lax.axis_index("s") * NUM_CORES + lax.axis_index("c")
    base = wid * IDX_PER_W
    pltpu.sync_copy(idx_hbm.at[pl.ds(base, IDX_PER_W)], idx_v)
    pltpu.async_copy(table_hbm.at[idx_v], rows_v, sem).wait()

    def row_body(r, _):
        rb = r * CTX
        for c in range(EMB_CHUNKS):
            s = rows_v[rb, pl.ds(c * LANES, LANES)]
            for t in range(1, CTX):
                s = s + rows_v[rb + t, pl.ds(c * LANES, LANES)]
            acc_v[r, pl.ds(c * LANES, LANES)] = s
        return 0

    lax.fori_loop(0, B_PER_W, row_body, 0)
    pltpu.sync_copy(acc_v, out_hbm.at[pl.ds(wid * B_PER_W, B_PER_W)])


@functools.cache
def _gather_sum_kernel():
    return pl.kernel(
        _gather_sum_body,
        mesh=plsc.VectorSubcoreMesh(core_axis_name="c", subcore_axis_name="s"),
        out_type=jax.ShapeDtypeStruct((B, EMB), jnp.float32),
        scratch_types=[
            pltpu.VMEM((IDX_PER_W,), jnp.int32),
            pltpu.VMEM((IDX_PER_W, EMB), jnp.float32),
            pltpu.VMEM((B_PER_W, EMB), jnp.float32),
            pltpu.SemaphoreType.DMA,
        ],
        compiler_params=pltpu.CompilerParams(use_tc_tiling_on_sc=False),
    )


def _mlp_body(sum_ref, wh_ref, bh_ref, wo_ref, bo_ref, out_ref, ht_ref):
    @pl.when(pl.program_id(0) == 0)
    def _():
        avg = sum_ref[...] * (1.0 / CTX)
        ht = lax.dot_general(wh_ref[...], avg, (((1,), (1,)), ((), ())),
                             preferred_element_type=jnp.float32)
        ht_ref[...] = jnp.maximum(ht + bh_ref[...], 0.0)

    logits_t = lax.dot_general(wo_ref[...], ht_ref[...],
                               (((1,), (0,)), ((), ())),
                               preferred_element_type=jnp.float32)
    out_ref[...] = logits_t + bo_ref[...]


def _mlp(emb_sum, W_hidden, b_hidden, W_out, b_out):
    out_t = pl.pallas_call(
        _mlp_body,
        grid=(N_TILES,),
        in_specs=[
            pl.BlockSpec((B, EMB), lambda i: (0, 0)),
            pl.BlockSpec((HID, EMB), lambda i: (0, 0)),
            pl.BlockSpec((HID, 1), lambda i: (0, 0)),
            pl.BlockSpec((V_TILE, HID), lambda i: (i, 0)),
            pl.BlockSpec((V_TILE, 1), lambda i: (i, 0)),
        ],
        out_specs=pl.BlockSpec((V_TILE, B), lambda i: (i, 0)),
        out_shape=jax.ShapeDtypeStruct((VOCAB, B), jnp.float32),
        scratch_shapes=[pltpu.VMEM((HID, B), jnp.float32)],
    )(emb_sum, W_hidden, b_hidden.reshape(HID, 1), W_out,
      b_out.reshape(VOCAB, 1))
    return out_t.T


def kernel(context, emb_table, W_hidden, b_hidden, W_out, b_out):
    idx = context.reshape(-1).astype(jnp.int32)
    emb_sum = _gather_sum_kernel()(idx, emb_table)
    return _mlp(emb_sum, W_hidden, b_hidden, W_out, b_out)


# speedup vs baseline: 2.6074x; 1.2755x over previous
import functools

import jax
import jax.numpy as jnp
from jax import lax
from jax.experimental import pallas as pl
from jax.experimental.pallas import tpu as pltpu
from jax.experimental.pallas import tpu_sc as plsc

VOCAB = 100000
EMB = 64
HID = 128
B = 1024
CTX = 20

NUM_CORES = 2
NUM_SUBCORES = 16
NW = NUM_CORES * NUM_SUBCORES
B_PER_W = B // NW
IDX_PER_W = B_PER_W * CTX
LANES = 16
EMB_CHUNKS = EMB // LANES
EMB_PAD = 128

V_TILE = 2048
N_TILES = (VOCAB + V_TILE - 1) // V_TILE


def _gather_sum_body(idx_hbm, table_hbm, out_hbm, idx_v, rows_v, acc_v, sem):
    wid = lax.axis_index("s") * NUM_CORES + lax.axis_index("c")
    base = wid * IDX_PER_W
    pltpu.sync_copy(idx_hbm.at[pl.ds(base, IDX_PER_W)], idx_v)
    pltpu.async_copy(table_hbm.at[idx_v], rows_v, sem).wait()

    def row_body(r, _):
        rb = r * CTX
        for c in range(EMB_CHUNKS):
            s = rows_v[rb, pl.ds(c * LANES, LANES)]
            for t in range(1, CTX):
                s = s + rows_v[rb + t, pl.ds(c * LANES, LANES)]
            acc_v[r, pl.ds(c * LANES, LANES)] = s
        return 0

    lax.fori_loop(0, B_PER_W, row_body, 0)
    pltpu.sync_copy(acc_v, out_hbm.at[pl.ds(wid * B_PER_W, B_PER_W)])


@functools.cache
def _gather_sum_kernel():
    return pl.kernel(
        _gather_sum_body,
        mesh=plsc.VectorSubcoreMesh(core_axis_name="c", subcore_axis_name="s"),
        out_type=jax.ShapeDtypeStruct((B, EMB_PAD), jnp.float32),
        scratch_types=[
            pltpu.VMEM((IDX_PER_W,), jnp.int32),
            pltpu.VMEM((IDX_PER_W, EMB_PAD), jnp.float32),
            pltpu.VMEM((B_PER_W, EMB_PAD), jnp.float32),
            pltpu.SemaphoreType.DMA,
        ],
    )


def _mlp_body(sum_ref, wh_ref, bh_ref, wo_ref, bo_ref, out_ref, ht_ref):
    @pl.when(pl.program_id(0) == 0)
    def _():
        avg = sum_ref[...][:, :EMB] * (1.0 / CTX)
        ht = lax.dot_general(wh_ref[...], avg, (((1,), (1,)), ((), ())),
                             preferred_element_type=jnp.float32)
        ht_ref[...] = jnp.maximum(ht + bh_ref[...], 0.0)

    logits_t = lax.dot_general(wo_ref[...], ht_ref[...],
                               (((1,), (0,)), ((), ())),
                               preferred_element_type=jnp.float32)
    bias = lax.dot_general(bo_ref[...], jnp.ones((1, B), jnp.float32),
                           (((0,), (0,)), ((), ())),
                           preferred_element_type=jnp.float32)
    out_ref[...] = logits_t + bias


def _mlp(emb_sum, W_hidden, b_hidden, W_out, b_out):
    out_t = pl.pallas_call(
        _mlp_body,
        grid=(N_TILES,),
        in_specs=[
            pl.BlockSpec((B, EMB_PAD), lambda i: (0, 0)),
            pl.BlockSpec((HID, EMB), lambda i: (0, 0)),
            pl.BlockSpec((HID, 1), lambda i: (0, 0)),
            pl.BlockSpec((V_TILE, HID), lambda i: (i, 0)),
            pl.BlockSpec((1, V_TILE), lambda i: (0, i)),
        ],
        out_specs=pl.BlockSpec((V_TILE, B), lambda i: (i, 0)),
        out_shape=jax.ShapeDtypeStruct((VOCAB, B), jnp.float32),
        scratch_shapes=[pltpu.VMEM((HID, B), jnp.float32)],
    )(emb_sum, W_hidden, b_hidden.reshape(HID, 1), W_out,
      b_out.reshape(1, VOCAB))
    return out_t.T


def kernel(context, emb_table, W_hidden, b_hidden, W_out, b_out):
    idx = context.reshape(-1).astype(jnp.int32)
    table_pad = jnp.pad(emb_table, ((0, 0), (0, EMB_PAD - EMB)))
    emb_sum = _gather_sum_kernel()(idx, table_pad)
    return _mlp(emb_sum, W_hidden, b_hidden, W_out, b_out)


# speedup vs baseline: 2.6446x; 1.0142x over previous
import functools

import jax
import jax.numpy as jnp
from jax import lax
from jax.experimental import pallas as pl
from jax.experimental.pallas import tpu as pltpu
from jax.experimental.pallas import tpu_sc as plsc

VOCAB = 100000
EMB = 64
HID = 128
B = 1024
CTX = 20

NUM_CORES = 2
NUM_SUBCORES = 16
NW = NUM_CORES * NUM_SUBCORES
B_PER_W = B // NW
IDX_PER_W = B_PER_W * CTX
LANES = 16
EMB_CHUNKS = EMB // LANES
EMB_PAD = 128

V_TILE = 4096
N_TILES = (VOCAB + V_TILE - 1) // V_TILE


def _gather_sum_body(idx_hbm, table_hbm, out_hbm, idx_v, rows_v, acc_v, sem):
    wid = lax.axis_index("s") * NUM_CORES + lax.axis_index("c")
    base = wid * IDX_PER_W
    pltpu.sync_copy(idx_hbm.at[pl.ds(base, IDX_PER_W)], idx_v)
    pltpu.async_copy(table_hbm.at[idx_v], rows_v, sem).wait()

    def row_body(r, _):
        rb = r * CTX
        for c in range(EMB_CHUNKS):
            s = rows_v[rb, pl.ds(c * LANES, LANES)]
            for t in range(1, CTX):
                s = s + rows_v[rb + t, pl.ds(c * LANES, LANES)]
            acc_v[r, pl.ds(c * LANES, LANES)] = s
        return 0

    lax.fori_loop(0, B_PER_W, row_body, 0)
    pltpu.sync_copy(acc_v, out_hbm.at[pl.ds(wid * B_PER_W, B_PER_W)])


@functools.cache
def _gather_sum_kernel():
    return pl.kernel(
        _gather_sum_body,
        mesh=plsc.VectorSubcoreMesh(core_axis_name="c", subcore_axis_name="s"),
        out_type=jax.ShapeDtypeStruct((B, EMB_PAD), jnp.float32),
        scratch_types=[
            pltpu.VMEM((IDX_PER_W,), jnp.int32),
            pltpu.VMEM((IDX_PER_W, EMB_PAD), jnp.float32),
            pltpu.VMEM((B_PER_W, EMB_PAD), jnp.float32),
            pltpu.SemaphoreType.DMA,
        ],
    )


def _mlp_body(sum_ref, wh_ref, bh_ref, wo_ref, bo_ref, out_ref, ht_ref):
    @pl.when(pl.program_id(0) == 0)
    def _():
        avg = sum_ref[...][:, :EMB] * (1.0 / CTX)
        ht = lax.dot_general(wh_ref[...], avg, (((1,), (1,)), ((), ())),
                             preferred_element_type=jnp.float32)
        ht_ref[...] = jnp.maximum(ht + bh_ref[...], 0.0)

    logits_t = lax.dot_general(wo_ref[...], ht_ref[...],
                               (((1,), (0,)), ((), ())),
                               preferred_element_type=jnp.float32)
    bias = lax.dot_general(bo_ref[...], jnp.ones((1, B), jnp.float32),
                           (((0,), (0,)), ((), ())),
                           preferred_element_type=jnp.float32)
    out_ref[...] = logits_t + bias


def _mlp(emb_sum, W_hidden, b_hidden, W_out, b_out):
    out_t = pl.pallas_call(
        _mlp_body,
        grid=(N_TILES,),
        in_specs=[
            pl.BlockSpec((B, EMB_PAD), lambda i: (0, 0)),
            pl.BlockSpec((HID, EMB), lambda i: (0, 0)),
            pl.BlockSpec((HID, 1), lambda i: (0, 0)),
            pl.BlockSpec((V_TILE, HID), lambda i: (i, 0)),
            pl.BlockSpec((1, V_TILE), lambda i: (0, i)),
        ],
        out_specs=pl.BlockSpec((V_TILE, B), lambda i: (i, 0)),
        out_shape=jax.ShapeDtypeStruct((VOCAB, B), jnp.float32),
        scratch_shapes=[pltpu.VMEM((HID, B), jnp.float32)],
    )(emb_sum, W_hidden, b_hidden.reshape(HID, 1), W_out,
      b_out.reshape(1, VOCAB))
    return out_t.T


def kernel(context, emb_table, W_hidden, b_hidden, W_out, b_out):
    idx = context.reshape(-1).astype(jnp.int32)
    table_pad = jnp.pad(emb_table, ((0, 0), (0, EMB_PAD - EMB)))
    emb_sum = _gather_sum_kernel()(idx, table_pad)
    return _mlp(emb_sum, W_hidden, b_hidden, W_out, b_out)
